# baseline (speedup 1.0000x reference)
import functools

import jax
import jax.numpy as jnp
from jax import lax
from jax.experimental import pallas as pl
from jax.experimental.pallas import tpu as pltpu
from jax.experimental.pallas import tpu_sc as plsc

DIM = 64
B_TOTAL = 4096 * 200
_info = plsc.get_sparse_core_info()
NC, NS = _info.num_cores, _info.num_subcores
NW = NC * NS
B_PER_W = B_TOTAL // NW
CHUNK = 800
NCHUNK = B_PER_W // CHUNK

_mesh = plsc.VectorSubcoreMesh(core_axis_name="c", subcore_axis_name="s")


@functools.partial(
    pl.kernel,
    mesh=_mesh,
    out_type=jax.ShapeDtypeStruct((B_TOTAL, DIM), jnp.float32),
    scratch_types=[
        pltpu.VMEM((CHUNK,), jnp.int32),
        pltpu.VMEM((CHUNK, DIM), jnp.float32),
        pltpu.SemaphoreType.DMA,
    ],
    compiler_params=pltpu.CompilerParams(use_tc_tiling_on_sc=False),
)
def _emb_lookup(x_hbm, table_hbm, out_hbm, idx_v, rows_v, sem):
    wid = lax.axis_index("s") * NC + lax.axis_index("c")
    base = wid * B_PER_W

    def body(i, carry):
        off = base + i * CHUNK
        pltpu.sync_copy(x_hbm.at[pl.ds(off, CHUNK)], idx_v)
        pltpu.async_copy(table_hbm.at[idx_v], rows_v, sem).wait()
        pltpu.sync_copy(rows_v, out_hbm.at[pl.ds(off, CHUNK)])
        return carry

    lax.fori_loop(0, NCHUNK, body, 0)


def kernel(x, table):
    xf = x.reshape(-1).astype(jnp.int32)
    out = _emb_lookup(xf, table)
    return out.reshape(x.shape + (table.shape[1],))


# speedup vs baseline: 1.0215x; 1.0215x over previous
import functools

import jax
import jax.numpy as jnp
from jax import lax
from jax.experimental import pallas as pl
from jax.experimental.pallas import tpu as pltpu
from jax.experimental.pallas import tpu_sc as plsc

DIM = 64
B_TOTAL = 4096 * 200
_info = plsc.get_sparse_core_info()
NC, NS = _info.num_cores, _info.num_subcores
NW = NC * NS
B_PER_W = B_TOTAL // NW
CHUNK = 800
NCHUNK = B_PER_W // CHUNK
K = 2
NROUND = NCHUNK // K

_mesh = plsc.VectorSubcoreMesh(core_axis_name="c", subcore_axis_name="s")


@functools.partial(
    pl.kernel,
    mesh=_mesh,
    out_type=jax.ShapeDtypeStruct((B_TOTAL, DIM), jnp.float32),
    scratch_types=[
        pltpu.VMEM((K, CHUNK), jnp.int32),
        pltpu.VMEM((K, CHUNK, DIM), jnp.float32),
    ]
    + [pltpu.SemaphoreType.DMA] * (3 * K),
    compiler_params=pltpu.CompilerParams(use_tc_tiling_on_sc=False),
)
def _emb_lookup(x_hbm, table_hbm, out_hbm, idx_v, rows_v, *sems):
    sem_i = sems[0:K]
    sem_g = sems[K : 2 * K]
    sem_s = sems[2 * K : 3 * K]
    wid = lax.axis_index("s") * NC + lax.axis_index("c")
    base = wid * B_PER_W

    for b in range(K):
        pltpu.async_copy(
            x_hbm.at[pl.ds(base + b * CHUNK, CHUNK)], idx_v.at[b], sem_i[b]
        )

    def body(g, carry):
        for b in range(K):
            @pl.when(g > 0)
            def _wait_store(b=b):
                pltpu.make_async_copy(
                    rows_v.at[b], out_hbm.at[pl.ds(base, CHUNK)], sem_s[b]
                ).wait()

            pltpu.make_async_copy(
                x_hbm.at[pl.ds(base, CHUNK)], idx_v.at[b], sem_i[b]
            ).wait()
            pltpu.async_copy(table_hbm.at[idx_v.at[b]], rows_v.at[b], sem_g[b])

        for b in range(K):
            off = base + (g * K + b) * CHUNK
            pltpu.make_async_copy(
                table_hbm.at[idx_v.at[b]], rows_v.at[b], sem_g[b]
            ).wait()
            pltpu.async_copy(rows_v.at[b], out_hbm.at[pl.ds(off, CHUNK)], sem_s[b])

            @pl.when(g < NROUND - 1)
            def _prefetch_idx(b=b, off=off):
                pltpu.async_copy(
                    x_hbm.at[pl.ds(off + K * CHUNK, CHUNK)], idx_v.at[b], sem_i[b]
                )

        return carry

    lax.fori_loop(0, NROUND, body, 0)

    for b in range(K):
        pltpu.make_async_copy(
            rows_v.at[b], out_hbm.at[pl.ds(base, CHUNK)], sem_s[b]
        ).wait()


def kernel(x, table):
    xf = x.reshape(-1).astype(jnp.int32)
    out = _emb_lookup(xf, table)
    return out.reshape(x.shape + (table.shape[1],))


# speedup vs baseline: 1.0266x; 1.0051x over previous
import functools

import jax
import jax.numpy as jnp
from jax import lax
from jax.experimental import pallas as pl
from jax.experimental.pallas import tpu as pltpu
from jax.experimental.pallas import tpu_sc as plsc

VOCAB = 1000000
DIM = 64
B = 4096
T = 200
_info = plsc.get_sparse_core_info()
NC, NS = _info.num_cores, _info.num_subcores
NW = NC * NS
ROWS_PER_W = B // NW
K = 4
NROUND = ROWS_PER_W // K

_mesh = plsc.VectorSubcoreMesh(core_axis_name="c", subcore_axis_name="s")


@functools.partial(
    pl.kernel,
    mesh=_mesh,
    out_type=jax.ShapeDtypeStruct((B, T, DIM), jnp.float32),
    scratch_types=[
        pltpu.VMEM((K, T), jnp.int32),
        pltpu.VMEM((K, T, DIM), jnp.float32),
    ]
    + [pltpu.SemaphoreType.DMA] * (3 * K),
    compiler_params=pltpu.CompilerParams(use_tc_tiling_on_sc=False),
)
def _emb_lookup(x_hbm, table_hbm, out_hbm, idx_v, rows_v, *sems):
    sem_i = sems[0:K]
    sem_g = sems[K : 2 * K]
    sem_s = sems[2 * K : 3 * K]
    wid = lax.axis_index("s") * NC + lax.axis_index("c")
    row0 = wid * ROWS_PER_W

    for b in range(K):
        pltpu.async_copy(x_hbm.at[pl.ds(row0 * T + b * T, T)], idx_v.at[b], sem_i[b])

    def body(g, carry):
        for b in range(K):
            @pl.when(g > 0)
            def _wait_store(b=b):
                pltpu.make_async_copy(rows_v.at[b], out_hbm.at[row0], sem_s[b]).wait()

            pltpu.make_async_copy(
                x_hbm.at[pl.ds(row0 * T, T)], idx_v.at[b], sem_i[b]
            ).wait()
            pltpu.async_copy(table_hbm.at[idx_v.at[b]], rows_v.at[b], sem_g[b])

        for b in range(K):
            row = row0 + g * K + b
            pltpu.make_async_copy(
                table_hbm.at[idx_v.at[b]], rows_v.at[b], sem_g[b]
            ).wait()
            pltpu.async_copy(rows_v.at[b], out_hbm.at[row], sem_s[b])

            @pl.when(g < NROUND - 1)
            def _prefetch_idx(b=b, row=row):
                pltpu.async_copy(
                    x_hbm.at[pl.ds((row + K) * T, T)], idx_v.at[b], sem_i[b]
                )

        return carry

    lax.fori_loop(0, NROUND, body, 0)

    for b in range(K):
        pltpu.make_async_copy(rows_v.at[b], out_hbm.at[row0], sem_s[b]).wait()


def kernel(x, table):
    xf = x.reshape(-1).astype(jnp.int32)
    return _emb_lookup(xf, table)
